# baseline (speedup 1.0000x reference)
import functools

import jax
import jax.numpy as jnp
from jax import lax
from jax.experimental import pallas as pl
from jax.experimental.pallas import tpu as pltpu
from jax.experimental.pallas import tpu_sc as plsc

_LANES = 16
_NBINS = 10001
_NB_PAD = 10240
_CHUNK = 16384
_UNROLL = 16


def _phase1(preds, targets):
    n = preds.shape[0]
    info = plsc.get_sparse_core_info()
    nc, ns = info.num_cores, info.num_subcores
    nw = nc * ns
    per_tile = n // nw
    nchunks = per_tile // _CHUNK
    slice_w = _NB_PAD // ns
    mesh = plsc.VectorSubcoreMesh(core_axis_name="c", subcore_axis_name="s")

    @functools.partial(
        pl.kernel,
        out_type=[
            jax.ShapeDtypeStruct((nc, 2, _NB_PAD), jnp.float32),
            jax.ShapeDtypeStruct((nc * 2 * ns * _LANES,), jnp.float32),
        ],
        mesh=mesh,
        compiler_params=pltpu.CompilerParams(needs_layout_passes=False),
        scratch_types=[
            pltpu.VMEM((_CHUNK,), jnp.float32),
            pltpu.VMEM((_CHUNK,), jnp.float32),
            pltpu.VMEM((_CHUNK,), jnp.float32),
            pltpu.VMEM((_CHUNK,), jnp.float32),
            pltpu.VMEM((2 * _NB_PAD,), jnp.float32),
            pltpu.VMEM_SHARED((ns, 2 * _NB_PAD), jnp.float32),
            pltpu.VMEM((ns, slice_w), jnp.float32),
            pltpu.VMEM((ns, slice_w), jnp.float32),
            pltpu.VMEM((slice_w,), jnp.float32),
            pltpu.VMEM((slice_w,), jnp.float32),
            pltpu.VMEM((_LANES,), jnp.float32),
            pltpu.SemaphoreType.DMA,
            pltpu.SemaphoreType.DMA,
            pltpu.SemaphoreType.DMA,
            pltpu.SemaphoreType.DMA,
            pltpu.SemaphoreType.DMA,
        ],
    )
    def k(preds_hbm, targets_hbm, out_hbm, tots_hbm, pbuf0, pbuf1, tbuf0,
          tbuf1, hist, shared, gneg, gpos, aneg, apos, tmp,
          psem0, psem1, tsem0, tsem1, gsem):
        cid = lax.axis_index("c")
        sid = lax.axis_index("s")
        wid = sid * nc + cid

        pbufs = (pbuf0, pbuf1)
        tbufs = (tbuf0, tbuf1)
        psems = (psem0, psem1)
        tsems = (tsem0, tsem1)

        zeros = jnp.zeros((_LANES,), jnp.float32)
        ones = jnp.ones((_LANES,), jnp.float32)

        with jax.named_scope("p1_zero"):
            @plsc.parallel_loop(0, 2 * _NB_PAD // _LANES, unroll=8)
            def _(i):
                hist[pl.ds(i * _LANES, _LANES)] = zeros

        base = wid * per_tile

        for b in range(2):
            off = base + b * _CHUNK
            pltpu.async_copy(preds_hbm.at[pl.ds(off, _CHUNK)], pbufs[b], psems[b])
            pltpu.async_copy(targets_hbm.at[pl.ds(off, _CHUNK)], tbufs[b], tsems[b])

        def cbody(jj, _):
            for b in range(2):
                j = jj * 2 + b
                pb, tb = pbufs[b], tbufs[b]
                pltpu.make_async_copy(
                    preds_hbm.at[pl.ds(0, _CHUNK)], pb, psems[b]).wait()
                pltpu.make_async_copy(
                    targets_hbm.at[pl.ds(0, _CHUNK)], tb, tsems[b]).wait()

                @plsc.parallel_loop(0, _CHUNK // _LANES, unroll=_UNROLL)
                def _(i, pb=pb, tb=tb):
                    ds = pl.ds(i * _LANES, _LANES)
                    p = pb[ds]
                    t = tb[ds]
                    s = 1.0 / (1.0 + jnp.exp(-p))
                    bn = (10000.0 * s).astype(jnp.int32)
                    half = jnp.where(t >= 0.5, _NB_PAD, 0)
                    plsc.addupdate_scatter(hist, [bn + half], ones)

                nxt = j + 2

                @pl.when(nxt < nchunks)
                def _(b=b, pb=pb, tb=tb, nxt=nxt):
                    off = base + nxt * _CHUNK
                    pltpu.async_copy(
                        preds_hbm.at[pl.ds(off, _CHUNK)], pb, psems[b])
                    pltpu.async_copy(
                        targets_hbm.at[pl.ds(off, _CHUNK)], tb, tsems[b])
            return 0

        with jax.named_scope("p1_hist"):
            lax.fori_loop(0, nchunks // 2, cbody, 0)

        with jax.named_scope("p1_stage"):
            pltpu.sync_copy(hist, shared.at[sid])
        with jax.named_scope("p1_barrier"):
            plsc.subcore_barrier()

        colbase = sid * slice_w
        with jax.named_scope("p1_gather"):
            for t in range(ns):
                pltpu.async_copy(
                    shared.at[t, pl.ds(colbase, slice_w)], gneg.at[t], gsem)
                pltpu.async_copy(
                    shared.at[t, pl.ds(_NB_PAD + colbase, slice_w)], gpos.at[t],
                    gsem)
            for t in range(ns):
                pltpu.make_async_copy(
                    shared.at[t, pl.ds(colbase, slice_w)], gneg.at[t], gsem).wait()
                pltpu.make_async_copy(
                    shared.at[t, pl.ds(colbase, slice_w)], gpos.at[t], gsem).wait()

        with jax.named_scope("p1_reduce"):
            @plsc.parallel_loop(0, slice_w // _LANES, unroll=4)
            def _(v):
                ds = pl.ds(v * _LANES, _LANES)
                sn = gneg[0, ds]
                sp = gpos[0, ds]
                for t in range(1, ns):
                    sn = sn + gneg[t, ds]
                    sp = sp + gpos[t, ds]
                aneg[ds] = sn
                apos[ds] = sp

        with jax.named_scope("p1_out"):
            pltpu.sync_copy(aneg, out_hbm.at[cid, 0, pl.ds(colbase, slice_w)])
            pltpu.sync_copy(apos, out_hbm.at[cid, 1, pl.ds(colbase, slice_w)])

        def sbody(v, carry):
            sn, sp = carry
            ds = pl.ds(v * _LANES, _LANES)
            return (sn + aneg[ds], sp + apos[ds])

        sn, sp = lax.fori_loop(0, slice_w // _LANES, sbody, (zeros, zeros))
        tmp[...] = jnp.broadcast_to(jnp.sum(sn), (_LANES,))
        pltpu.sync_copy(
            tmp, tots_hbm.at[pl.ds(((cid * 2 + 0) * ns + sid) * _LANES, _LANES)])
        tmp[...] = jnp.broadcast_to(jnp.sum(sp), (_LANES,))
        pltpu.sync_copy(
            tmp, tots_hbm.at[pl.ds(((cid * 2 + 1) * ns + sid) * _LANES, _LANES)])

    return k(preds, targets)


def _phase2(part, tots):
    nc = part.shape[0]
    info = plsc.get_sparse_core_info()
    ns = info.num_subcores
    slice_w = _NB_PAD // ns
    nv = slice_w // _LANES
    mesh = plsc.VectorSubcoreMesh(core_axis_name="c", subcore_axis_name="s")

    @functools.partial(
        pl.kernel,
        out_type=[
            jax.ShapeDtypeStruct((_LANES,), jnp.float32),
            jax.ShapeDtypeStruct((ns * _LANES,), jnp.float32),
        ],
        mesh=mesh,
        compiler_params=pltpu.CompilerParams(needs_layout_passes=False),
        scratch_types=[
            pltpu.VMEM((nc, 2, slice_w), jnp.float32),
            pltpu.VMEM((slice_w,), jnp.float32),
            pltpu.VMEM((slice_w,), jnp.float32),
            pltpu.VMEM((nc * 2 * ns * _LANES,), jnp.float32),
            pltpu.VMEM((ns * _LANES,), jnp.float32),
            pltpu.VMEM((_LANES,), jnp.float32),
            pltpu.VMEM((_LANES,), jnp.float32),
        ],
    )
    def k(part_hbm, tots_hbm, out_hbm, mx_hbm, vbuf, fsl, tsl, ttot,
          gmx, tmp, obuf):
        cid = lax.axis_index("c")
        sid = lax.axis_index("s")

        @pl.when(cid == 0)
        def _():
            colbase = sid * slice_w
            for c in range(nc):
                for h in range(2):
                    pltpu.sync_copy(
                        part_hbm.at[c, h, pl.ds(colbase, slice_w)],
                        vbuf.at[c, h])
            pltpu.sync_copy(tots_hbm, ttot)

            zeros = jnp.zeros((_LANES,), jnp.float32)

            @plsc.parallel_loop(0, nv, unroll=4)
            def _(i):
                ds = pl.ds(i * _LANES, _LANES)
                fp = vbuf[0, 0, ds]
                tp = vbuf[0, 1, ds]
                for c in range(1, nc):
                    fp = fp + vbuf[c, 0, ds]
                    tp = tp + vbuf[c, 1, ds]
                fsl[ds] = fp
                tsl[ds] = tp

            sidv = jnp.broadcast_to(sid, (_LANES,))
            pref_t = zeros
            pref_f = zeros
            tot_t = zeros
            tot_f = zeros
            for c in range(nc):
                for j in range(ns):
                    rowf = ttot[pl.ds(((c * 2 + 0) * ns + j) * _LANES, _LANES)]
                    rowt = ttot[pl.ds(((c * 2 + 1) * ns + j) * _LANES, _LANES)]
                    before = jnp.broadcast_to(jnp.int32(j), (_LANES,)) < sidv
                    pref_f = pref_f + jnp.where(before, rowf, zeros)
                    pref_t = pref_t + jnp.where(before, rowt, zeros)
                    tot_f = tot_f + rowf
                    tot_t = tot_t + rowt

            inv_p = 1.0 / tot_t
            inv_n = 1.0 / tot_f

            def kbody(i, carry):
                ct, cf, m = carry
                ds = pl.ds(i * _LANES, _LANES)
                tp = tsl[ds]
                fp = fsl[ds]
                tpc = plsc.cumsum(tp) + ct
                fpc = plsc.cumsum(fp) + cf
                d = jnp.abs(tpc * inv_p - fpc * inv_n)
                m = jnp.maximum(m, jnp.max(d))
                return (ct + jnp.sum(tp), cf + jnp.sum(fp), m)

            _, _, m = lax.fori_loop(0, nv, kbody, (pref_t, pref_f, 0.0))

            tmp[...] = jnp.broadcast_to(m, (_LANES,))
            pltpu.sync_copy(tmp, mx_hbm.at[pl.ds(sid * _LANES, _LANES)])
            plsc.subcore_barrier()

            @pl.when(sid == 0)
            def _():
                pltpu.sync_copy(mx_hbm, gmx)
                mm = gmx[pl.ds(0, _LANES)]
                for j in range(1, ns):
                    mm = jnp.maximum(mm, gmx[pl.ds(j * _LANES, _LANES)])
                obuf[...] = mm
                pltpu.sync_copy(obuf, out_hbm)

    return k(part, tots)


def kernel(preds, targets):
    part, tots = _phase1(preds, targets)
    ks, _ = _phase2(part, tots)
    return ks[0]


# speedup vs baseline: 1.0882x; 1.0882x over previous
import functools

import jax
import jax.numpy as jnp
from jax import lax
from jax.experimental import pallas as pl
from jax.experimental.pallas import tpu as pltpu
from jax.experimental.pallas import tpu_sc as plsc

_LANES = 16
_NBINS = 10001
_NB_PAD = 10240
_CHUNK = 16384
_UNROLL = 16


def _phase1(preds, targets):
    n = preds.shape[0]
    info = plsc.get_sparse_core_info()
    nc, ns = info.num_cores, info.num_subcores
    nw = nc * ns
    per_tile = n // nw
    nchunks = per_tile // _CHUNK
    slice_w = _NB_PAD // ns
    mesh = plsc.VectorSubcoreMesh(core_axis_name="c", subcore_axis_name="s")

    @functools.partial(
        pl.kernel,
        out_type=[
            jax.ShapeDtypeStruct((nc, 2, _NB_PAD), jnp.float32),
            jax.ShapeDtypeStruct((nc * 2 * ns * _LANES,), jnp.float32),
        ],
        mesh=mesh,
        compiler_params=pltpu.CompilerParams(needs_layout_passes=False),
        scratch_types=[
            pltpu.VMEM((_CHUNK,), jnp.float32),
            pltpu.VMEM((_CHUNK,), jnp.float32),
            pltpu.VMEM((_CHUNK,), jnp.float32),
            pltpu.VMEM((_CHUNK,), jnp.float32),
            pltpu.VMEM((2 * _NB_PAD,), jnp.float32),
            pltpu.VMEM_SHARED((ns, 2 * _NB_PAD), jnp.float32),
            pltpu.VMEM((ns, slice_w), jnp.float32),
            pltpu.VMEM((ns, slice_w), jnp.float32),
            pltpu.VMEM((slice_w,), jnp.float32),
            pltpu.VMEM((slice_w,), jnp.float32),
            pltpu.VMEM((_LANES,), jnp.float32),
            pltpu.SemaphoreType.DMA,
            pltpu.SemaphoreType.DMA,
            pltpu.SemaphoreType.DMA,
            pltpu.SemaphoreType.DMA,
            pltpu.SemaphoreType.DMA,
        ],
    )
    def k(preds_hbm, targets_hbm, out_hbm, tots_hbm, pbuf0, pbuf1, tbuf0,
          tbuf1, hist, shared, gneg, gpos, aneg, apos, tmp,
          psem0, psem1, tsem0, tsem1, gsem):
        cid = lax.axis_index("c")
        sid = lax.axis_index("s")
        wid = sid * nc + cid

        pbufs = (pbuf0, pbuf1)
        tbufs = (tbuf0, tbuf1)
        psems = (psem0, psem1)
        tsems = (tsem0, tsem1)

        zeros = jnp.zeros((_LANES,), jnp.float32)
        ones = jnp.ones((_LANES,), jnp.float32)

        base = wid * per_tile

        for b in range(2):
            off = base + b * _CHUNK
            pltpu.async_copy(preds_hbm.at[pl.ds(off, _CHUNK)], pbufs[b], psems[b])
            pltpu.async_copy(targets_hbm.at[pl.ds(off, _CHUNK)], tbufs[b], tsems[b])

        with jax.named_scope("p1_zero"):
            @plsc.parallel_loop(0, 2 * _NB_PAD // _LANES, unroll=8)
            def _(i):
                hist[pl.ds(i * _LANES, _LANES)] = zeros

        def cbody(jj, _):
            for b in range(2):
                j = jj * 2 + b
                pb, tb = pbufs[b], tbufs[b]
                pltpu.make_async_copy(
                    preds_hbm.at[pl.ds(0, _CHUNK)], pb, psems[b]).wait()
                pltpu.make_async_copy(
                    targets_hbm.at[pl.ds(0, _CHUNK)], tb, tsems[b]).wait()

                @plsc.parallel_loop(0, _CHUNK // _LANES, unroll=_UNROLL)
                def _(i, pb=pb, tb=tb):
                    ds = pl.ds(i * _LANES, _LANES)
                    p = pb[ds]
                    t = tb[ds]
                    s = 1.0 / (1.0 + jnp.exp(-p))
                    bn = (10000.0 * s).astype(jnp.int32)
                    half = jnp.where(t >= 0.5, _NB_PAD, 0)
                    plsc.addupdate_scatter(hist, [bn + half], ones)

                nxt = j + 2

                @pl.when(nxt < nchunks)
                def _(b=b, pb=pb, tb=tb, nxt=nxt):
                    off = base + nxt * _CHUNK
                    pltpu.async_copy(
                        preds_hbm.at[pl.ds(off, _CHUNK)], pb, psems[b])
                    pltpu.async_copy(
                        targets_hbm.at[pl.ds(off, _CHUNK)], tb, tsems[b])
            return 0

        with jax.named_scope("p1_hist"):
            lax.fori_loop(0, nchunks // 2, cbody, 0)

        with jax.named_scope("p1_stage"):
            pltpu.sync_copy(hist, shared.at[sid])
        with jax.named_scope("p1_barrier"):
            plsc.subcore_barrier()

        colbase = sid * slice_w
        with jax.named_scope("p1_gather"):
            for t in range(ns):
                pltpu.async_copy(
                    shared.at[t, pl.ds(colbase, slice_w)], gneg.at[t], gsem)
                pltpu.async_copy(
                    shared.at[t, pl.ds(_NB_PAD + colbase, slice_w)], gpos.at[t],
                    gsem)
            for t in range(ns):
                pltpu.make_async_copy(
                    shared.at[t, pl.ds(colbase, slice_w)], gneg.at[t], gsem).wait()
                pltpu.make_async_copy(
                    shared.at[t, pl.ds(colbase, slice_w)], gpos.at[t], gsem).wait()

        with jax.named_scope("p1_reduce"):
            @plsc.parallel_loop(0, slice_w // _LANES, unroll=4)
            def _(v):
                ds = pl.ds(v * _LANES, _LANES)
                sn = gneg[0, ds]
                sp = gpos[0, ds]
                for t in range(1, ns):
                    sn = sn + gneg[t, ds]
                    sp = sp + gpos[t, ds]
                aneg[ds] = sn
                apos[ds] = sp

        with jax.named_scope("p1_out"):
            pltpu.sync_copy(aneg, out_hbm.at[cid, 0, pl.ds(colbase, slice_w)])
            pltpu.sync_copy(apos, out_hbm.at[cid, 1, pl.ds(colbase, slice_w)])

        def sbody(v, carry):
            sn, sp = carry
            ds = pl.ds(v * _LANES, _LANES)
            return (sn + aneg[ds], sp + apos[ds])

        sn, sp = lax.fori_loop(0, slice_w // _LANES, sbody, (zeros, zeros))
        tmp[...] = jnp.broadcast_to(jnp.sum(sn), (_LANES,))
        pltpu.sync_copy(
            tmp, tots_hbm.at[pl.ds(((cid * 2 + 0) * ns + sid) * _LANES, _LANES)])
        tmp[...] = jnp.broadcast_to(jnp.sum(sp), (_LANES,))
        pltpu.sync_copy(
            tmp, tots_hbm.at[pl.ds(((cid * 2 + 1) * ns + sid) * _LANES, _LANES)])

    return k(preds, targets)


def _phase2(part, tots):
    nc = part.shape[0]
    info = plsc.get_sparse_core_info()
    ns = info.num_subcores
    slice_w = _NB_PAD // ns
    nv = slice_w // _LANES
    mesh = plsc.VectorSubcoreMesh(core_axis_name="c", subcore_axis_name="s")

    @functools.partial(
        pl.kernel,
        out_type=[
            jax.ShapeDtypeStruct((_LANES,), jnp.float32),
            jax.ShapeDtypeStruct((ns * _LANES,), jnp.float32),
        ],
        mesh=mesh,
        compiler_params=pltpu.CompilerParams(needs_layout_passes=False),
        scratch_types=[
            pltpu.VMEM((nc, 2, slice_w), jnp.float32),
            pltpu.VMEM((slice_w,), jnp.float32),
            pltpu.VMEM((slice_w,), jnp.float32),
            pltpu.VMEM((nc * 2 * ns * _LANES,), jnp.float32),
            pltpu.VMEM((ns * _LANES,), jnp.float32),
            pltpu.VMEM((_LANES,), jnp.float32),
            pltpu.VMEM((_LANES,), jnp.float32),
        ],
    )
    def k(part_hbm, tots_hbm, out_hbm, mx_hbm, vbuf, fsl, tsl, ttot,
          gmx, tmp, obuf):
        cid = lax.axis_index("c")
        sid = lax.axis_index("s")

        @pl.when(cid == 0)
        def _():
            colbase = sid * slice_w
            for c in range(nc):
                for h in range(2):
                    pltpu.sync_copy(
                        part_hbm.at[c, h, pl.ds(colbase, slice_w)],
                        vbuf.at[c, h])
            pltpu.sync_copy(tots_hbm, ttot)

            zeros = jnp.zeros((_LANES,), jnp.float32)

            @plsc.parallel_loop(0, nv, unroll=4)
            def _(i):
                ds = pl.ds(i * _LANES, _LANES)
                fp = vbuf[0, 0, ds]
                tp = vbuf[0, 1, ds]
                for c in range(1, nc):
                    fp = fp + vbuf[c, 0, ds]
                    tp = tp + vbuf[c, 1, ds]
                fsl[ds] = fp
                tsl[ds] = tp

            sidv = jnp.broadcast_to(sid, (_LANES,))
            pref_t = zeros
            pref_f = zeros
            tot_t = zeros
            tot_f = zeros
            for c in range(nc):
                for j in range(ns):
                    rowf = ttot[pl.ds(((c * 2 + 0) * ns + j) * _LANES, _LANES)]
                    rowt = ttot[pl.ds(((c * 2 + 1) * ns + j) * _LANES, _LANES)]
                    before = jnp.broadcast_to(jnp.int32(j), (_LANES,)) < sidv
                    pref_f = pref_f + jnp.where(before, rowf, zeros)
                    pref_t = pref_t + jnp.where(before, rowt, zeros)
                    tot_f = tot_f + rowf
                    tot_t = tot_t + rowt

            inv_p = 1.0 / tot_t
            inv_n = 1.0 / tot_f

            def kbody(i, carry):
                ct, cf, m = carry
                ds = pl.ds(i * _LANES, _LANES)
                tp = tsl[ds]
                fp = fsl[ds]
                tpc = plsc.cumsum(tp) + ct
                fpc = plsc.cumsum(fp) + cf
                d = jnp.abs(tpc * inv_p - fpc * inv_n)
                m = jnp.maximum(m, jnp.max(d))
                return (ct + jnp.sum(tp), cf + jnp.sum(fp), m)

            _, _, m = lax.fori_loop(0, nv, kbody, (pref_t, pref_f, 0.0))

            tmp[...] = jnp.broadcast_to(m, (_LANES,))
            pltpu.sync_copy(tmp, mx_hbm.at[pl.ds(sid * _LANES, _LANES)])
            plsc.subcore_barrier()

            @pl.when(sid == 0)
            def _():
                pltpu.sync_copy(mx_hbm, gmx)
                mm = gmx[pl.ds(0, _LANES)]
                for j in range(1, ns):
                    mm = jnp.maximum(mm, gmx[pl.ds(j * _LANES, _LANES)])
                obuf[...] = mm
                pltpu.sync_copy(obuf, out_hbm)

    return k(part, tots)


def _phase2_tc(part4):
    nc = part4.shape[0]
    r, c = part4.shape[2], part4.shape[3]

    def _scan0(x, n):
        sh = 1
        while sh < n:
            z = jnp.zeros((sh,) + x.shape[1:], jnp.float32)
            x = x + jnp.concatenate([z, x[: n - sh]], axis=0)
            sh *= 2
        return x

    def _scan1(x, n):
        sh = 1
        while sh < n:
            z = jnp.zeros(x.shape[:1] + (sh,), jnp.float32)
            x = x + jnp.concatenate([z, x[:, : n - sh]], axis=1)
            sh *= 2
        return x

    def cum2d(x):
        colcum = _scan0(x, r)
        coltot = colcum[r - 1:r, :]
        colpref = _scan1(coltot, c) - coltot
        return colcum + colpref

    def body(p_ref, o_ref):
        fp = p_ref[0, 0]
        tp = p_ref[0, 1]
        for k in range(1, nc):
            fp = fp + p_ref[k, 0]
            tp = tp + p_ref[k, 1]
        cum_tp = cum2d(tp)
        cum_fp = cum2d(fp)
        tot_tp = jnp.sum(tp)
        tot_fp = jnp.sum(fp)
        d = jnp.abs(cum_tp / tot_tp - cum_fp / tot_fp)
        o_ref[...] = jnp.broadcast_to(jnp.max(d), (1, 1))

    return pl.pallas_call(
        body,
        out_shape=jax.ShapeDtypeStruct((1, 1), jnp.float32),
    )(part4)


def kernel(preds, targets):
    part, tots = _phase1(preds, targets)
    part4 = part.reshape(part.shape[0], 2, 128, 80).swapaxes(2, 3)
    ks = _phase2_tc(part4)
    return ks[0, 0]
